# baseline (speedup 1.0000x reference)
import jax
import jax.numpy as jnp
from jax.experimental import pallas as pl
from jax.experimental.pallas import tpu as pltpu

B = 8
NUM_SLOTS = 8192
SLOT_DIM = 64
N_HEADS = 8
TOPK = 16
BOTTLENECK = 64

_SQRT2 = 1.4142135623730951
_HI = jax.lax.Precision.HIGHEST
_NCHUNK = 4


def _dot(a, b, dims, precision=None):
    return jax.lax.dot_general(a, b, (dims, ((), ())), precision=precision,
                               preferred_element_type=jnp.float32)


def _body(mem_ref, keys_ref, vals_ref, erase_ref, addg_ref, beta_ref,
          w1_ref, b1_ref, w2_ref, b2_ref, decay_ref, age_ref,
          newmem_ref, w_out_ref):
    b = pl.program_id(0)
    keys = keys_ref[b]
    vals = vals_ref[b]
    erase = erase_ref[b]
    addg = addg_ref[b]
    beta = beta_ref[b]

    h = _dot(vals, w1_ref[...], ((1,), (0,))) + b1_ref[...]
    h = 0.5 * h * (1.0 + jax.lax.erf(h / _SQRT2))
    cv = _dot(h, w2_ref[...], ((1,), (0,))) + b2_ref[...]
    cvg = cv * (addg * (1.0 / N_HEADS))

    kn = keys / jnp.maximum(
        jnp.sqrt(jnp.sum(keys * keys, axis=1, keepdims=True)), 1e-12)
    ones_row = jnp.ones((1, SLOT_DIM), jnp.float32)

    CH = NUM_SLOTS // _NCHUNK
    sims = []
    for c in range(_NCHUNK):
        msl = mem_ref[0, pl.ds(c * CH, CH), :]
        s_c = _dot(ones_row, msl * msl, ((1,), (1,)), _HI)
        n_c = jnp.maximum(jnp.sqrt(s_c), 1e-12)
        mn_c = msl / jnp.reshape(n_c, (CH, 1))
        sims.append(_dot(kn, mn_c, ((1,), (1,))))
    sim = jnp.concatenate(sims, axis=1)
    a1 = age_ref[...] + 1.0
    ab = a1 * (1.0 / (jnp.max(a1) + 1e-8))
    sim = sim * beta + ab

    iota = jax.lax.broadcasted_iota(jnp.int32, (N_HEADS, NUM_SLOTS), 1)
    work = sim
    mask = jnp.zeros((N_HEADS, NUM_SLOTS), dtype=jnp.bool_)
    m0 = jnp.max(work, axis=1, keepdims=True)
    for _ in range(TOPK):
        m = jnp.max(work, axis=1, keepdims=True)
        cand = jnp.where(work == m, iota, NUM_SLOTS)
        first = jnp.min(cand, axis=1, keepdims=True)
        sel = iota == first
        mask = jnp.logical_or(mask, sel)
        work = jnp.where(sel, -jnp.inf, work)

    wexp = jnp.where(mask, jnp.exp(sim - m0), 0.0)
    w = wexp / jnp.sum(wexp, axis=1, keepdims=True)
    w_out_ref[0] = w

    er = erase * (1.0 / N_HEADS)
    s_news = []
    for c in range(_NCHUNK):
        sl = pl.ds(c * CH, CH)
        msl = mem_ref[0, sl, :]
        w_c = w[:, c * CH:(c + 1) * CH]
        e_c = _dot(w_c, er, ((0,), (0,)))
        a_c = _dot(w_c, cvg, ((0,), (0,)))
        new_c = msl - msl * e_c + a_c + 1e-8
        s_news.append(_dot(ones_row, new_c * new_c, ((1,), (1,))))
        newmem_ref[0, sl, :] = new_c
    s_new = jnp.concatenate(s_news, axis=1)
    scale_row = (jax.nn.sigmoid(decay_ref[...])
                 / jnp.maximum(jnp.sqrt(s_new), 1e-12))
    for c in range(_NCHUNK):
        sl = pl.ds(c * CH, CH)
        sc_c = jnp.reshape(scale_row[:, c * CH:(c + 1) * CH], (CH, 1))
        newmem_ref[0, sl, :] = newmem_ref[0, sl, :] * sc_c


@jax.jit
def kernel(memory, write_keys, write_vals, erase, add_gate, beta,
           W1, b1, W2, b2, decay_gate, age):
    full = lambda s: pl.BlockSpec(s, lambda b: tuple(0 for _ in s))
    grid_spec = pl.GridSpec(
        grid=(B,),
        in_specs=[
            pl.BlockSpec((1, NUM_SLOTS, SLOT_DIM), lambda b: (b, 0, 0)),
            full((B, N_HEADS, SLOT_DIM)),
            full((B, N_HEADS, SLOT_DIM)),
            full((B, N_HEADS, 1)),
            full((B, N_HEADS, 1)),
            full((B, N_HEADS, 1)),
            full((SLOT_DIM, BOTTLENECK)),
            full((1, BOTTLENECK)),
            full((BOTTLENECK, SLOT_DIM)),
            full((1, SLOT_DIM)),
            full((1, NUM_SLOTS)),
            full((1, NUM_SLOTS)),
        ],
        out_specs=[
            pl.BlockSpec((1, NUM_SLOTS, SLOT_DIM), lambda b: (b, 0, 0)),
            pl.BlockSpec((1, N_HEADS, NUM_SLOTS), lambda b: (b, 0, 0)),
        ],
    )
    new_memory, weights = pl.pallas_call(
        _body,
        grid_spec=grid_spec,
        compiler_params=pltpu.CompilerParams(
            dimension_semantics=("parallel",)),
        out_shape=[
            jax.ShapeDtypeStruct((B, NUM_SLOTS, SLOT_DIM), jnp.float32),
            jax.ShapeDtypeStruct((B, N_HEADS, NUM_SLOTS), jnp.float32),
        ],
    )(memory, write_keys, write_vals,
      erase[..., None], add_gate[..., None], beta[..., None],
      W1, b1.reshape(1, BOTTLENECK), W2, b2.reshape(1, SLOT_DIM),
      decay_gate.reshape(1, NUM_SLOTS), age)
    return (new_memory, weights)


# speedup vs baseline: 1.0461x; 1.0461x over previous
import jax
import jax.numpy as jnp
from jax.experimental import pallas as pl
from jax.experimental.pallas import tpu as pltpu

B = 8
NUM_SLOTS = 8192
SLOT_DIM = 64
N_HEADS = 8
TOPK = 16
BOTTLENECK = 64

_SQRT2 = 1.4142135623730951
_HI = jax.lax.Precision.HIGHEST
_NCHUNK = 4


def _dot(a, b, dims, precision=None):
    return jax.lax.dot_general(a, b, (dims, ((), ())), precision=precision,
                               preferred_element_type=jnp.float32)


def _body(mem_ref, keys_ref, vals_ref, erase_ref, addg_ref, beta_ref,
          w1_ref, b1_ref, w2_ref, b2_ref, decay_ref, age_ref,
          newmem_ref, w_out_ref):
    b = pl.program_id(0)
    keys = keys_ref[b]
    vals = vals_ref[b]
    erase = erase_ref[b]
    addg = addg_ref[b]
    beta = beta_ref[b]

    h = _dot(vals, w1_ref[...], ((1,), (0,))) + b1_ref[...]
    h = 0.5 * h * (1.0 + jax.lax.erf(h / _SQRT2))
    cv = _dot(h, w2_ref[...], ((1,), (0,))) + b2_ref[...]
    cvg = cv * (addg * (1.0 / N_HEADS))

    kn = keys / jnp.maximum(
        jnp.sqrt(jnp.sum(keys * keys, axis=1, keepdims=True)), 1e-12)
    ones_row = jnp.ones((1, SLOT_DIM), jnp.float32)

    mem = mem_ref[0]
    s_mem = _dot(ones_row, mem * mem, ((1,), (1,)), _HI)
    n_row = jnp.maximum(jnp.sqrt(s_mem), 1e-12)
    mem_n = mem / jnp.reshape(n_row, (NUM_SLOTS, 1))
    sim = _dot(kn, mem_n, ((1,), (1,)))
    a1 = age_ref[...] + 1.0
    ab = a1 * (1.0 / (jnp.max(a1) + 1e-8))
    sim = sim * beta + ab

    iota = jax.lax.broadcasted_iota(jnp.int32, (N_HEADS, NUM_SLOTS), 1)
    work = sim
    mask = jnp.zeros((N_HEADS, NUM_SLOTS), dtype=jnp.bool_)
    m0 = jnp.max(work, axis=1, keepdims=True)
    for _ in range(TOPK):
        m = jnp.max(work, axis=1, keepdims=True)
        cand = jnp.where(work == m, iota, NUM_SLOTS)
        first = jnp.min(cand, axis=1, keepdims=True)
        sel = iota == first
        mask = jnp.logical_or(mask, sel)
        work = jnp.where(sel, -jnp.inf, work)

    wexp = jnp.where(mask, jnp.exp(sim - m0), 0.0)
    w = wexp / jnp.sum(wexp, axis=1, keepdims=True)
    w_out_ref[0] = w

    e_col = _dot(w, erase * (1.0 / N_HEADS), ((0,), (0,)))
    a_mat = _dot(w, cvg, ((0,), (0,)))
    new = mem - mem * e_col + a_mat + 1e-8
    s_new = _dot(ones_row, new * new, ((1,), (1,)))
    scale_row = (jax.nn.sigmoid(decay_ref[...])
                 / jnp.maximum(jnp.sqrt(s_new), 1e-12))
    newmem_ref[0] = new * jnp.reshape(scale_row, (NUM_SLOTS, 1))


@jax.jit
def kernel(memory, write_keys, write_vals, erase, add_gate, beta,
           W1, b1, W2, b2, decay_gate, age):
    full = lambda s: pl.BlockSpec(s, lambda b: tuple(0 for _ in s))
    grid_spec = pl.GridSpec(
        grid=(B,),
        in_specs=[
            pl.BlockSpec((1, NUM_SLOTS, SLOT_DIM), lambda b: (b, 0, 0)),
            full((B, N_HEADS, SLOT_DIM)),
            full((B, N_HEADS, SLOT_DIM)),
            full((B, N_HEADS, 1)),
            full((B, N_HEADS, 1)),
            full((B, N_HEADS, 1)),
            full((SLOT_DIM, BOTTLENECK)),
            full((1, BOTTLENECK)),
            full((BOTTLENECK, SLOT_DIM)),
            full((1, SLOT_DIM)),
            full((1, NUM_SLOTS)),
            full((1, NUM_SLOTS)),
        ],
        out_specs=[
            pl.BlockSpec((1, NUM_SLOTS, SLOT_DIM), lambda b: (b, 0, 0)),
            pl.BlockSpec((1, N_HEADS, NUM_SLOTS), lambda b: (b, 0, 0)),
        ],
    )
    new_memory, weights = pl.pallas_call(
        _body,
        grid_spec=grid_spec,
        compiler_params=pltpu.CompilerParams(
            dimension_semantics=("parallel",)),
        out_shape=[
            jax.ShapeDtypeStruct((B, NUM_SLOTS, SLOT_DIM), jnp.float32),
            jax.ShapeDtypeStruct((B, N_HEADS, NUM_SLOTS), jnp.float32),
        ],
    )(memory, write_keys, write_vals,
      erase[..., None], add_gate[..., None], beta[..., None],
      W1, b1.reshape(1, BOTTLENECK), W2, b2.reshape(1, SLOT_DIM),
      decay_gate.reshape(1, NUM_SLOTS), age)
    return (new_memory, weights)
